# baseline (speedup 1.0000x reference)
import jax
import jax.numpy as jnp
import numpy as np
from jax.experimental import pallas as pl


def _knot_maps():
    M = np.zeros((12, 10), np.float32)
    C = np.zeros((12, 1), np.float32)
    M[0, 0] = -1.0; M[0, 1] = -1.0; C[0, 0] = -10000.0
    M[1, 0] = -1.0; M[1, 1] = -1.0
    M[2, 1] = -1.0
    M[3, 2] = 1.0
    M[4, 2] = 1.0; M[4, 3] = 1.0
    M[5, 2] = 1.0; M[5, 3] = 1.0; C[5, 0] = 10000.0
    M[6, 4] = -1.0; M[6, 5] = -1.0; M[6, 8] = -20000.0
    M[7, 4] = -1.0; M[7, 5] = -1.0
    M[8, 5] = -1.0
    M[9, 6] = 1.0
    M[10, 6] = 1.0; M[10, 7] = 1.0
    M[11, 6] = 1.0; M[11, 7] = 1.0; M[11, 9] = 20000.0
    return jnp.asarray(M), jnp.asarray(C)


def _interleave_maps():
    S = np.zeros((384, 384), np.float32)
    T = np.zeros((384, 384), np.float32)
    for l in range(128):
        for c in range(3):
            S[3 * l + c, 128 * c + l] = 1.0
            T[128 * c + l, 3 * l + c] = 1.0
    return jnp.asarray(S), jnp.asarray(T)


def _body(xw_ref, t_ref, mask_ref, W1zT_ref, W1tT_ref, b1_ref, W2T_ref,
          b2_ref, M_ref, C_ref, S_ref, T_ref, out_ref):
    bi, bl = xw_ref.shape
    b = bi * bl // 3
    planes = xw_ref[...] @ S_ref[...]
    x0 = planes[:, 0:128].reshape(1, b)
    x1 = planes[:, 128:256].reshape(1, b)
    qx0 = planes[:, 256:384].reshape(1, b)
    zT = jnp.tanh(jnp.concatenate([x0, x1], axis=0))
    tT = jnp.transpose(t_ref[...])
    hT = W1zT_ref[...] @ zT + W1tT_ref[...] @ tT + b1_ref[...]
    hT = jnp.maximum(hT, 0.0)
    pT = W2T_ref[...] @ hT + b2_ref[...]
    dT = jax.nn.softplus(pT) + 1e-4
    axy = M_ref[...] @ dT + C_ref[...]
    ax = axy[0:6, :]
    ay = axy[6:12, :]
    qx = jnp.clip(qx0, ax[0:1, :] * 0.99, ax[5:6, :] * 0.99)
    xl = ax[0:5, :]
    xr = ax[1:6, :]
    yl = ay[0:5, :]
    yr = ay[1:6, :]
    sel = ((qx >= xl) & (qx < xr)).astype(jnp.float32)
    cand = (yr - yl) / (xr - xl) * (qx - xl) + yl
    gy = jnp.sum(cand * sel, axis=0, keepdims=True)
    yp = jnp.concatenate([
        (x0 * mask_ref[0:1, :]).reshape(bi, 128),
        (x1 * mask_ref[1:2, :]).reshape(bi, 128),
        gy.reshape(bi, 128),
    ], axis=1)
    out_ref[...] = yp @ T_ref[...]


def kernel(x, t_feat, mask, W1, b1, W2, b2):
    n, p, _ = x.shape
    rows = n * p
    blk = 2048
    bi = blk // p
    grid = rows // blk
    M, C = _knot_maps()
    S, T = _interleave_maps()
    const = lambda i: (0, 0)
    out = pl.pallas_call(
        _body,
        grid=(grid,),
        in_specs=[
            pl.BlockSpec((bi, 3 * p), lambda i: (i, 0)),
            pl.BlockSpec((blk, 64), lambda i: (i, 0)),
            pl.BlockSpec((3, 1), const),
            pl.BlockSpec((128, 2), const),
            pl.BlockSpec((128, 64), const),
            pl.BlockSpec((128, 1), const),
            pl.BlockSpec((10, 128), const),
            pl.BlockSpec((10, 1), const),
            pl.BlockSpec((12, 10), const),
            pl.BlockSpec((12, 1), const),
            pl.BlockSpec((384, 384), const),
            pl.BlockSpec((384, 384), const),
        ],
        out_specs=pl.BlockSpec((bi, 3 * p), lambda i: (i, 0)),
        out_shape=jax.ShapeDtypeStruct((n, 3 * p), jnp.float32),
    )(x.reshape(n, 3 * p), t_feat.reshape(rows, 64), mask.reshape(3, 1),
      W1[0:2, :].T, W1[2:, :].T, b1.reshape(128, 1),
      W2.T, b2.reshape(10, 1), M, C, S, T)
    return out.reshape(n, p, 3)


# speedup vs baseline: 1.6830x; 1.6830x over previous
import jax
import jax.numpy as jnp
import numpy as np
from jax.experimental import pallas as pl


def _knot_maps():
    M = np.zeros((12, 10), np.float32)
    C = np.zeros((12, 1), np.float32)
    M[0, 0] = -1.0; M[0, 1] = -1.0; C[0, 0] = -10000.0
    M[1, 0] = -1.0; M[1, 1] = -1.0
    M[2, 1] = -1.0
    M[3, 2] = 1.0
    M[4, 2] = 1.0; M[4, 3] = 1.0
    M[5, 2] = 1.0; M[5, 3] = 1.0; C[5, 0] = 10000.0
    M[6, 4] = -1.0; M[6, 5] = -1.0; M[6, 8] = -20000.0
    M[7, 4] = -1.0; M[7, 5] = -1.0
    M[8, 5] = -1.0
    M[9, 6] = 1.0
    M[10, 6] = 1.0; M[10, 7] = 1.0
    M[11, 6] = 1.0; M[11, 7] = 1.0; M[11, 9] = 20000.0
    return jnp.asarray(M), jnp.asarray(C)


def _body(x_ref, t_ref, mask_ref, W1zT_ref, W1tT_ref, b1_ref, W2T_ref,
          b2_ref, M_ref, C_ref, out_ref):
    _, bn, p = x_ref.shape
    b = bn * p
    x3 = x_ref[...]
    z2 = jnp.tanh(x3[0:2]).reshape(2, b)
    tT = jnp.transpose(t_ref[...], (1, 0, 2)).reshape(64, b)
    hT = W1zT_ref[...] @ z2 + W1tT_ref[...] @ tT + b1_ref[...]
    hT = jnp.maximum(hT, 0.0)
    pT = W2T_ref[...] @ hT + b2_ref[...]
    dT = jax.nn.softplus(pT) + 1e-4
    axy = M_ref[...] @ dT + C_ref[...]
    ax = axy[0:6, :]
    ay = axy[6:12, :]
    qx0 = x3[2].reshape(1, b)
    qx = jnp.clip(qx0, ax[0:1, :] * 0.99, ax[5:6, :] * 0.99)
    xl = ax[0:5, :]
    xr = ax[1:6, :]
    yl = ay[0:5, :]
    yr = ay[1:6, :]
    sel = ((qx >= xl) & (qx < xr)).astype(jnp.float32)
    cand = (yr - yl) / (xr - xl) * (qx - xl) + yl
    gy = jnp.sum(cand * sel, axis=0, keepdims=True)
    out_ref[...] = jnp.concatenate(
        [x3[0:2] * mask_ref[0:2].reshape(2, 1, 1), gy.reshape(1, bn, p)],
        axis=0)


def kernel(x, t_feat, mask, W1, b1, W2, b2):
    n, p, _ = x.shape
    rows = n * p
    blk = 2048
    bn = blk // p
    grid = rows // blk
    M, C = _knot_maps()
    x3 = jnp.transpose(x, (2, 0, 1))
    tt = jnp.transpose(t_feat, (0, 2, 1))
    const = lambda i: (0, 0)
    out = pl.pallas_call(
        _body,
        grid=(grid,),
        in_specs=[
            pl.BlockSpec((3, bn, p), lambda i: (0, i, 0)),
            pl.BlockSpec((bn, 64, p), lambda i: (i, 0, 0)),
            pl.BlockSpec((3, 1), const),
            pl.BlockSpec((128, 2), const),
            pl.BlockSpec((128, 64), const),
            pl.BlockSpec((128, 1), const),
            pl.BlockSpec((10, 128), const),
            pl.BlockSpec((10, 1), const),
            pl.BlockSpec((12, 10), const),
            pl.BlockSpec((12, 1), const),
        ],
        out_specs=pl.BlockSpec((3, bn, p), lambda i: (0, i, 0)),
        out_shape=jax.ShapeDtypeStruct((3, n, p), jnp.float32),
    )(x3, tt, mask.reshape(3, 1),
      W1[0:2, :].T, W1[2:, :].T, b1.reshape(128, 1),
      W2.T, b2.reshape(10, 1), M, C)
    return jnp.transpose(out, (1, 2, 0))


# speedup vs baseline: 2.8054x; 1.6669x over previous
import jax
import jax.numpy as jnp
import numpy as np
from jax.experimental import pallas as pl


def _knot_maps():
    M = np.zeros((12, 10), np.float32)
    C = np.zeros((12, 1), np.float32)
    M[0, 0] = -1.0; M[0, 1] = -1.0; C[0, 0] = -10000.0
    M[1, 0] = -1.0; M[1, 1] = -1.0
    M[2, 1] = -1.0
    M[3, 2] = 1.0
    M[4, 2] = 1.0; M[4, 3] = 1.0
    M[5, 2] = 1.0; M[5, 3] = 1.0; C[5, 0] = 10000.0
    M[6, 4] = -1.0; M[6, 5] = -1.0; M[6, 8] = -20000.0
    M[7, 4] = -1.0; M[7, 5] = -1.0
    M[8, 5] = -1.0
    M[9, 6] = 1.0
    M[10, 6] = 1.0; M[10, 7] = 1.0
    M[11, 6] = 1.0; M[11, 7] = 1.0; M[11, 9] = 20000.0
    return jnp.asarray(M), jnp.asarray(C)


def _body(x_ref, t_ref, mask_ref, W1zT_ref, W1tT_ref, b1_ref, W2T_ref,
          b2_ref, M_ref, C_ref, out_ref):
    _, bn, p = x_ref.shape
    b = bn * p
    x3 = x_ref[...]
    z2 = jnp.tanh(x3[0:2]).reshape(2, b)
    tT = jnp.transpose(t_ref[...], (1, 0, 2)).reshape(64, b)
    hT = W1zT_ref[...] @ z2 + W1tT_ref[...] @ tT + b1_ref[...]
    hT = jnp.maximum(hT, 0.0)
    pT = W2T_ref[...] @ hT + b2_ref[...]
    dT = jax.nn.softplus(pT) + 1e-4
    axy = M_ref[...] @ dT + C_ref[...]
    ax = axy[0:6, :]
    ay = axy[6:12, :]
    qx0 = x3[2].reshape(1, b)
    qx = jnp.clip(qx0, ax[0:1, :] * 0.99, ax[5:6, :] * 0.99)
    xl = ax[0:5, :]
    xr = ax[1:6, :]
    yl = ay[0:5, :]
    yr = ay[1:6, :]
    sel = ((qx >= xl) & (qx < xr)).astype(jnp.float32)
    cand = (yr - yl) / (xr - xl) * (qx - xl) + yl
    gy = jnp.sum(cand * sel, axis=0, keepdims=True)
    out_ref[...] = jnp.concatenate(
        [x3[0:2] * mask_ref[0:2].reshape(2, 1, 1), gy.reshape(1, bn, p)],
        axis=0)


def kernel(x, t_feat, mask, W1, b1, W2, b2):
    n, p, _ = x.shape
    rows = n * p
    blk = 8192
    bn = blk // p
    grid = rows // blk
    M, C = _knot_maps()
    x3 = jnp.transpose(x, (2, 0, 1))
    tt = jnp.transpose(t_feat, (0, 2, 1))
    const = lambda i: (0, 0)
    out = pl.pallas_call(
        _body,
        grid=(grid,),
        in_specs=[
            pl.BlockSpec((3, bn, p), lambda i: (0, i, 0)),
            pl.BlockSpec((bn, 64, p), lambda i: (i, 0, 0)),
            pl.BlockSpec((3, 1), const),
            pl.BlockSpec((128, 2), const),
            pl.BlockSpec((128, 64), const),
            pl.BlockSpec((128, 1), const),
            pl.BlockSpec((10, 128), const),
            pl.BlockSpec((10, 1), const),
            pl.BlockSpec((12, 10), const),
            pl.BlockSpec((12, 1), const),
        ],
        out_specs=pl.BlockSpec((3, bn, p), lambda i: (0, i, 0)),
        out_shape=jax.ShapeDtypeStruct((3, n, p), jnp.float32),
    )(x3, tt, mask.reshape(3, 1),
      W1[0:2, :].T, W1[2:, :].T, b1.reshape(128, 1),
      W2.T, b2.reshape(10, 1), M, C)
    return jnp.transpose(out, (1, 2, 0))


# speedup vs baseline: 2.9661x; 1.0573x over previous
import jax
import jax.numpy as jnp
import numpy as np
from jax.experimental import pallas as pl


def _knot_maps():
    M = np.zeros((12, 10), np.float32)
    C = np.zeros((12, 1), np.float32)
    M[0, 0] = -1.0; M[0, 1] = -1.0; C[0, 0] = -10000.0
    M[1, 0] = -1.0; M[1, 1] = -1.0
    M[2, 1] = -1.0
    M[3, 2] = 1.0
    M[4, 2] = 1.0; M[4, 3] = 1.0
    M[5, 2] = 1.0; M[5, 3] = 1.0; C[5, 0] = 10000.0
    M[6, 4] = -1.0; M[6, 5] = -1.0; M[6, 8] = -20000.0
    M[7, 4] = -1.0; M[7, 5] = -1.0
    M[8, 5] = -1.0
    M[9, 6] = 1.0
    M[10, 6] = 1.0; M[10, 7] = 1.0
    M[11, 6] = 1.0; M[11, 7] = 1.0; M[11, 9] = 20000.0
    return jnp.asarray(M), jnp.asarray(C)


def _body(x_ref, t_ref, mask_ref, W1zT_ref, W1tT_ref, b1_ref, W2T_ref,
          b2_ref, M_ref, C_ref, out_ref):
    _, bn, p = x_ref.shape
    b = bn * p
    x3 = x_ref[...]
    z2 = jnp.tanh(x3[0:2]).reshape(2, b)
    tT = jnp.transpose(t_ref[...], (1, 0, 2)).reshape(64, b)
    hT = W1zT_ref[...] @ z2 + W1tT_ref[...] @ tT + b1_ref[...]
    hT = jnp.maximum(hT, 0.0)
    pT = W2T_ref[...] @ hT + b2_ref[...]
    dT = jax.nn.softplus(pT) + 1e-4
    axy = M_ref[...] @ dT + C_ref[...]
    ax = axy[0:6, :]
    ay = axy[6:12, :]
    qx0 = x3[2].reshape(1, b)
    qx = jnp.clip(qx0, ax[0:1, :] * 0.99, ax[5:6, :] * 0.99)
    xl = ax[0:5, :]
    xr = ax[1:6, :]
    yl = ay[0:5, :]
    yr = ay[1:6, :]
    sel = ((qx >= xl) & (qx < xr)).astype(jnp.float32)
    cand = (yr - yl) / (xr - xl) * (qx - xl) + yl
    gy = jnp.sum(cand * sel, axis=0, keepdims=True)
    out_ref[...] = jnp.concatenate(
        [x3[0:2] * mask_ref[0:2].reshape(2, 1, 1), gy.reshape(1, bn, p)],
        axis=0)


def kernel(x, t_feat, mask, W1, b1, W2, b2):
    n, p, _ = x.shape
    rows = n * p
    blk = 16384
    bn = blk // p
    grid = rows // blk
    M, C = _knot_maps()
    x3 = jnp.transpose(x, (2, 0, 1))
    tt = jnp.transpose(t_feat, (0, 2, 1))
    const = lambda i: (0, 0)
    out = pl.pallas_call(
        _body,
        grid=(grid,),
        in_specs=[
            pl.BlockSpec((3, bn, p), lambda i: (0, i, 0)),
            pl.BlockSpec((bn, 64, p), lambda i: (i, 0, 0)),
            pl.BlockSpec((3, 1), const),
            pl.BlockSpec((128, 2), const),
            pl.BlockSpec((128, 64), const),
            pl.BlockSpec((128, 1), const),
            pl.BlockSpec((10, 128), const),
            pl.BlockSpec((10, 1), const),
            pl.BlockSpec((12, 10), const),
            pl.BlockSpec((12, 1), const),
        ],
        out_specs=pl.BlockSpec((3, bn, p), lambda i: (0, i, 0)),
        out_shape=jax.ShapeDtypeStruct((3, n, p), jnp.float32),
    )(x3, tt, mask.reshape(3, 1),
      W1[0:2, :].T, W1[2:, :].T, b1.reshape(128, 1),
      W2.T, b2.reshape(10, 1), M, C)
    return jnp.transpose(out, (1, 2, 0))


# speedup vs baseline: 3.0503x; 1.0284x over previous
import jax
import jax.numpy as jnp
import numpy as np
from jax.experimental import pallas as pl


def _knot_maps():
    M = np.zeros((12, 10), np.float32)
    C = np.zeros((12, 1), np.float32)
    M[0, 0] = -1.0; M[0, 1] = -1.0; C[0, 0] = -10000.0
    M[1, 0] = -1.0; M[1, 1] = -1.0
    M[2, 1] = -1.0
    M[3, 2] = 1.0
    M[4, 2] = 1.0; M[4, 3] = 1.0
    M[5, 2] = 1.0; M[5, 3] = 1.0; C[5, 0] = 10000.0
    M[6, 4] = -1.0; M[6, 5] = -1.0; M[6, 8] = -20000.0
    M[7, 4] = -1.0; M[7, 5] = -1.0
    M[8, 5] = -1.0
    M[9, 6] = 1.0
    M[10, 6] = 1.0; M[10, 7] = 1.0
    M[11, 6] = 1.0; M[11, 7] = 1.0; M[11, 9] = 20000.0
    return jnp.asarray(M), jnp.asarray(C)


def _body(x_ref, t_ref, mask_ref, W1zT_ref, W1tT_ref, b1_ref, W2T_ref,
          b2_ref, M_ref, C_ref, out_ref):
    _, bn, p = x_ref.shape
    b = bn * p
    x3 = x_ref[...]
    z2 = jnp.tanh(x3[0:2]).reshape(2, b)
    tT = jnp.transpose(t_ref[...], (1, 0, 2)).reshape(64, b)
    hT = W1zT_ref[...] @ z2 + W1tT_ref[...] @ tT + b1_ref[...]
    hT = jnp.maximum(hT, 0.0)
    pT = W2T_ref[...] @ hT + b2_ref[...]
    dT = jax.nn.softplus(pT) + 1e-4
    axy = M_ref[...] @ dT + C_ref[...]
    ax = axy[0:6, :]
    ay = axy[6:12, :]
    qx0 = x3[2].reshape(1, b)
    qx = jnp.clip(qx0, ax[0:1, :] * 0.99, ax[5:6, :] * 0.99)
    xl = ax[0:5, :]
    xr = ax[1:6, :]
    yl = ay[0:5, :]
    yr = ay[1:6, :]
    sel = ((qx >= xl) & (qx < xr)).astype(jnp.float32)
    cand = (yr - yl) / (xr - xl) * (qx - xl) + yl
    gy = jnp.sum(cand * sel, axis=0, keepdims=True)
    out_ref[...] = jnp.concatenate(
        [x3[0:2] * mask_ref[0:2].reshape(2, 1, 1), gy.reshape(1, bn, p)],
        axis=0)


def kernel(x, t_feat, mask, W1, b1, W2, b2):
    n, p, _ = x.shape
    rows = n * p
    blk = 32768
    bn = blk // p
    grid = rows // blk
    M, C = _knot_maps()
    x3 = jnp.transpose(x, (2, 0, 1))
    tt = jnp.transpose(t_feat, (0, 2, 1))
    const = lambda i: (0, 0)
    out = pl.pallas_call(
        _body,
        grid=(grid,),
        in_specs=[
            pl.BlockSpec((3, bn, p), lambda i: (0, i, 0)),
            pl.BlockSpec((bn, 64, p), lambda i: (i, 0, 0)),
            pl.BlockSpec((3, 1), const),
            pl.BlockSpec((128, 2), const),
            pl.BlockSpec((128, 64), const),
            pl.BlockSpec((128, 1), const),
            pl.BlockSpec((10, 128), const),
            pl.BlockSpec((10, 1), const),
            pl.BlockSpec((12, 10), const),
            pl.BlockSpec((12, 1), const),
        ],
        out_specs=pl.BlockSpec((3, bn, p), lambda i: (0, i, 0)),
        out_shape=jax.ShapeDtypeStruct((3, n, p), jnp.float32),
    )(x3, tt, mask.reshape(3, 1),
      W1[0:2, :].T, W1[2:, :].T, b1.reshape(128, 1),
      W2.T, b2.reshape(10, 1), M, C)
    return jnp.transpose(out, (1, 2, 0))


# speedup vs baseline: 3.5882x; 1.1764x over previous
import jax
import jax.numpy as jnp
import numpy as np
from jax.experimental import pallas as pl


def _knot_maps():
    M = np.zeros((12, 10), np.float32)
    C = np.zeros((12, 1), np.float32)
    M[0, 0] = -1.0; M[0, 1] = -1.0; C[0, 0] = -10000.0
    M[1, 0] = -1.0; M[1, 1] = -1.0
    M[2, 1] = -1.0
    M[3, 2] = 1.0
    M[4, 2] = 1.0; M[4, 3] = 1.0
    M[5, 2] = 1.0; M[5, 3] = 1.0; C[5, 0] = 10000.0
    M[6, 4] = -1.0; M[6, 5] = -1.0; M[6, 8] = -20000.0
    M[7, 4] = -1.0; M[7, 5] = -1.0
    M[8, 5] = -1.0
    M[9, 6] = 1.0
    M[10, 6] = 1.0; M[10, 7] = 1.0
    M[11, 6] = 1.0; M[11, 7] = 1.0; M[11, 9] = 20000.0
    return jnp.asarray(M), jnp.asarray(C)


def _body(x_ref, t_ref, mask_ref, W1zT_ref, W1tT_ref, b1_ref, W2T_ref,
          b2_ref, M_ref, C_ref, out_ref):
    _, bn, p = x_ref.shape
    b = bn * p
    x3 = x_ref[...]
    z2 = jnp.tanh(x3[0:2]).reshape(2, b)
    t_bf = t_ref[...].astype(jnp.bfloat16)
    tT_bf = jnp.transpose(t_bf, (1, 0, 2)).reshape(64, b)
    W1t_bf = W1tT_ref[...].astype(jnp.bfloat16)
    ht = jax.lax.dot_general(W1t_bf, tT_bf, (((1,), (0,)), ((), ())),
                             preferred_element_type=jnp.float32)
    hT = W1zT_ref[...] @ z2 + ht + b1_ref[...]
    hT = jnp.maximum(hT, 0.0)
    pT = (W2T_ref[...] @ hT).reshape(10, bn, p) + b2_ref[...].reshape(10, 1, 1)
    dT = jax.nn.softplus(pT) + 1e-4
    axy = (M_ref[...] @ dT.reshape(10, b)).reshape(12, bn, p) \
        + C_ref[...].reshape(12, 1, 1)
    ax = axy[0:6]
    ay = axy[6:12]
    qx = jnp.clip(x3[2:3], ax[0:1] * 0.99, ax[5:6] * 0.99)
    xl = ax[0:5]
    xr = ax[1:6]
    yl = ay[0:5]
    yr = ay[1:6]
    sel = ((qx >= xl) & (qx < xr)).astype(jnp.float32)
    xl_s = jnp.sum(xl * sel, axis=0, keepdims=True)
    xr_s = jnp.sum(xr * sel, axis=0, keepdims=True)
    yl_s = jnp.sum(yl * sel, axis=0, keepdims=True)
    yr_s = jnp.sum(yr * sel, axis=0, keepdims=True)
    gy = (yr_s - yl_s) / (xr_s - xl_s) * (qx - xl_s) + yl_s
    out_ref[...] = jnp.concatenate(
        [x3[0:2] * mask_ref[0:2].reshape(2, 1, 1), gy], axis=0)


def kernel(x, t_feat, mask, W1, b1, W2, b2):
    n, p, _ = x.shape
    rows = n * p
    blk = 32768
    bn = blk // p
    grid = rows // blk
    M, C = _knot_maps()
    x3 = jnp.transpose(x, (2, 0, 1))
    tt = jnp.transpose(t_feat, (0, 2, 1))
    const = lambda i: (0, 0)
    out = pl.pallas_call(
        _body,
        grid=(grid,),
        in_specs=[
            pl.BlockSpec((3, bn, p), lambda i: (0, i, 0)),
            pl.BlockSpec((bn, 64, p), lambda i: (i, 0, 0)),
            pl.BlockSpec((3, 1), const),
            pl.BlockSpec((128, 2), const),
            pl.BlockSpec((128, 64), const),
            pl.BlockSpec((128, 1), const),
            pl.BlockSpec((10, 128), const),
            pl.BlockSpec((10, 1), const),
            pl.BlockSpec((12, 10), const),
            pl.BlockSpec((12, 1), const),
        ],
        out_specs=pl.BlockSpec((3, bn, p), lambda i: (0, i, 0)),
        out_shape=jax.ShapeDtypeStruct((3, n, p), jnp.float32),
    )(x3, tt, mask.reshape(3, 1),
      W1[0:2, :].T, W1[2:, :].T, b1.reshape(128, 1),
      W2.T, b2.reshape(10, 1), M, C)
    return jnp.transpose(out, (1, 2, 0))


# speedup vs baseline: 3.9165x; 1.0915x over previous
import jax
import jax.numpy as jnp
import numpy as np
from jax.experimental import pallas as pl


def _knot_maps():
    M = np.zeros((12, 10), np.float32)
    C = np.zeros((12, 1), np.float32)
    M[0, 0] = -1.0; M[0, 1] = -1.0; C[0, 0] = -10000.0
    M[1, 0] = -1.0; M[1, 1] = -1.0
    M[2, 1] = -1.0
    M[3, 2] = 1.0
    M[4, 2] = 1.0; M[4, 3] = 1.0
    M[5, 2] = 1.0; M[5, 3] = 1.0; C[5, 0] = 10000.0
    M[6, 4] = -1.0; M[6, 5] = -1.0; M[6, 8] = -20000.0
    M[7, 4] = -1.0; M[7, 5] = -1.0
    M[8, 5] = -1.0
    M[9, 6] = 1.0
    M[10, 6] = 1.0; M[10, 7] = 1.0
    M[11, 6] = 1.0; M[11, 7] = 1.0; M[11, 9] = 20000.0
    return jnp.asarray(M), jnp.asarray(C)


def _body(x_ref, t_ref, mask_ref, W1a_ref, W2T_ref,
          b2_ref, M_ref, C_ref, out_ref):
    _, bn, p = x_ref.shape
    b = bn * p
    x3 = x_ref[...]
    z2 = jnp.tanh(x3[0:2]).astype(jnp.bfloat16).reshape(2, b)
    t_bf = t_ref[...].astype(jnp.bfloat16)
    tT_bf = jnp.transpose(t_bf, (1, 0, 2)).reshape(64, b)
    ones = jnp.ones((1, b), jnp.bfloat16)
    cat = jnp.concatenate([tT_bf, z2, ones], axis=0)
    W1a_bf = W1a_ref[...].astype(jnp.bfloat16)
    hT = jax.lax.dot_general(W1a_bf, cat, (((1,), (0,)), ((), ())),
                             preferred_element_type=jnp.float32)
    hT = jnp.maximum(hT, 0.0)
    pT = (W2T_ref[...] @ hT).reshape(10, bn, p) + b2_ref[...].reshape(10, 1, 1)
    dT = jax.nn.softplus(pT) + 1e-4
    axy = (M_ref[...] @ dT.reshape(10, b)).reshape(12, bn, p) \
        + C_ref[...].reshape(12, 1, 1)
    ax = axy[0:6]
    ay = axy[6:12]
    qx = jnp.clip(x3[2:3], ax[0:1] * 0.99, ax[5:6] * 0.99)
    xl = ax[0:5]
    xr = ax[1:6]
    yl = ay[0:5]
    yr = ay[1:6]
    sel = ((qx >= xl) & (qx < xr)).astype(jnp.float32)
    xl_s = jnp.sum(xl * sel, axis=0, keepdims=True)
    xr_s = jnp.sum(xr * sel, axis=0, keepdims=True)
    yl_s = jnp.sum(yl * sel, axis=0, keepdims=True)
    yr_s = jnp.sum(yr * sel, axis=0, keepdims=True)
    gy = (yr_s - yl_s) / (xr_s - xl_s) * (qx - xl_s) + yl_s
    out_ref[...] = jnp.concatenate(
        [x3[0:2] * mask_ref[0:2].reshape(2, 1, 1), gy], axis=0)


def kernel(x, t_feat, mask, W1, b1, W2, b2):
    n, p, _ = x.shape
    rows = n * p
    blk = 32768
    bn = blk // p
    grid = rows // blk
    M, C = _knot_maps()
    W1a = jnp.concatenate([W1[2:, :].T, W1[0:2, :].T, b1.reshape(128, 1)],
                          axis=1)
    x3 = jnp.transpose(x, (2, 0, 1))
    tt = jnp.transpose(t_feat, (0, 2, 1))
    const = lambda i: (0, 0)
    out = pl.pallas_call(
        _body,
        grid=(grid,),
        in_specs=[
            pl.BlockSpec((3, bn, p), lambda i: (0, i, 0)),
            pl.BlockSpec((bn, 64, p), lambda i: (i, 0, 0)),
            pl.BlockSpec((3, 1), const),
            pl.BlockSpec((128, 67), const),
            pl.BlockSpec((10, 128), const),
            pl.BlockSpec((10, 1), const),
            pl.BlockSpec((12, 10), const),
            pl.BlockSpec((12, 1), const),
        ],
        out_specs=pl.BlockSpec((3, bn, p), lambda i: (0, i, 0)),
        out_shape=jax.ShapeDtypeStruct((3, n, p), jnp.float32),
    )(x3, tt, mask.reshape(3, 1), W1a,
      W2.T, b2.reshape(10, 1), M, C)
    return jnp.transpose(out, (1, 2, 0))


# speedup vs baseline: 5.2984x; 1.3528x over previous
import jax
import jax.numpy as jnp
import numpy as np
from jax.experimental import pallas as pl


def _knot_maps():
    M = np.zeros((12, 10), np.float32)
    C = np.zeros((12, 1), np.float32)
    M[0, 0] = -1.0; M[0, 1] = -1.0; C[0, 0] = -10000.0
    M[1, 0] = -1.0; M[1, 1] = -1.0
    M[2, 1] = -1.0
    M[3, 2] = 1.0
    M[4, 2] = 1.0; M[4, 3] = 1.0
    M[5, 2] = 1.0; M[5, 3] = 1.0; C[5, 0] = 10000.0
    M[6, 4] = -1.0; M[6, 5] = -1.0; M[6, 8] = -20000.0
    M[7, 4] = -1.0; M[7, 5] = -1.0
    M[8, 5] = -1.0
    M[9, 6] = 1.0
    M[10, 6] = 1.0; M[10, 7] = 1.0
    M[11, 6] = 1.0; M[11, 7] = 1.0; M[11, 9] = 20000.0
    return jnp.asarray(M), jnp.asarray(C)


def _body(x_ref, t_ref, mask_ref, W1a_ref, W2T_ref,
          b2_ref, M_ref, C_ref, out_ref):
    _, bn, p = x_ref.shape
    b = bn * p
    x3 = x_ref[...]
    z2 = jnp.tanh(x3[0:2]).astype(jnp.bfloat16).reshape(2, b)
    t_bf = t_ref[...].astype(jnp.bfloat16)
    tT_bf = jnp.concatenate([t_bf[g] for g in range(bn)], axis=1)
    ones = jnp.ones((1, b), jnp.bfloat16)
    cat = jnp.concatenate([tT_bf, z2, ones], axis=0)
    W1a_bf = W1a_ref[...].astype(jnp.bfloat16)
    hT = jax.lax.dot_general(W1a_bf, cat, (((1,), (0,)), ((), ())),
                             preferred_element_type=jnp.float32)
    hT = jnp.maximum(hT, 0.0)
    pT = (W2T_ref[...] @ hT).reshape(10, bn, p) + b2_ref[...].reshape(10, 1, 1)
    dT = jax.nn.softplus(pT) + 1e-4
    axy = (M_ref[...] @ dT.reshape(10, b)).reshape(12, bn, p) \
        + C_ref[...].reshape(12, 1, 1)
    ax = axy[0:6]
    ay = axy[6:12]
    qx = jnp.clip(x3[2:3], ax[0:1] * 0.99, ax[5:6] * 0.99)
    xl = ax[0:5]
    xr = ax[1:6]
    yl = ay[0:5]
    yr = ay[1:6]
    sel = ((qx >= xl) & (qx < xr)).astype(jnp.float32)
    xl_s = jnp.sum(xl * sel, axis=0, keepdims=True)
    xr_s = jnp.sum(xr * sel, axis=0, keepdims=True)
    yl_s = jnp.sum(yl * sel, axis=0, keepdims=True)
    yr_s = jnp.sum(yr * sel, axis=0, keepdims=True)
    gy = (yr_s - yl_s) / (xr_s - xl_s) * (qx - xl_s) + yl_s
    out_ref[...] = jnp.concatenate(
        [x3[0:2] * mask_ref[0:2].reshape(2, 1, 1), gy], axis=0)


def kernel(x, t_feat, mask, W1, b1, W2, b2):
    n, p, _ = x.shape
    rows = n * p
    blk = 32768
    bn = blk // p
    grid = rows // blk
    M, C = _knot_maps()
    W1a = jnp.concatenate([W1[2:, :].T, W1[0:2, :].T, b1.reshape(128, 1)],
                          axis=1)
    x3 = jnp.transpose(x, (2, 0, 1))
    tt = jnp.transpose(t_feat, (0, 2, 1))
    const = lambda i: (0, 0)
    out = pl.pallas_call(
        _body,
        grid=(grid,),
        in_specs=[
            pl.BlockSpec((3, bn, p), lambda i: (0, i, 0)),
            pl.BlockSpec((bn, 64, p), lambda i: (i, 0, 0)),
            pl.BlockSpec((3, 1), const),
            pl.BlockSpec((128, 67), const),
            pl.BlockSpec((10, 128), const),
            pl.BlockSpec((10, 1), const),
            pl.BlockSpec((12, 10), const),
            pl.BlockSpec((12, 1), const),
        ],
        out_specs=pl.BlockSpec((3, bn, p), lambda i: (0, i, 0)),
        out_shape=jax.ShapeDtypeStruct((3, n, p), jnp.float32),
    )(x3, tt, mask.reshape(3, 1), W1a,
      W2.T, b2.reshape(10, 1), M, C)
    return jnp.transpose(out, (1, 2, 0))


# speedup vs baseline: 6.0227x; 1.1367x over previous
import jax
import jax.numpy as jnp
import numpy as np
from jax.experimental import pallas as pl


def _knot_maps():
    M = np.zeros((12, 10), np.float32)
    C = np.zeros((12, 1), np.float32)
    M[0, 0] = -1.0; M[0, 1] = -1.0; C[0, 0] = -10000.0
    M[1, 0] = -1.0; M[1, 1] = -1.0
    M[2, 1] = -1.0
    M[3, 2] = 1.0
    M[4, 2] = 1.0; M[4, 3] = 1.0
    M[5, 2] = 1.0; M[5, 3] = 1.0; C[5, 0] = 10000.0
    M[6, 4] = -1.0; M[6, 5] = -1.0; M[6, 8] = -20000.0
    M[7, 4] = -1.0; M[7, 5] = -1.0
    M[8, 5] = -1.0
    M[9, 6] = 1.0
    M[10, 6] = 1.0; M[10, 7] = 1.0
    M[11, 6] = 1.0; M[11, 7] = 1.0; M[11, 9] = 20000.0
    return jnp.asarray(M), jnp.asarray(C)


def _body(x_ref, t_ref, mask_ref, W1a_ref, W2T_ref,
          b2_ref, out_ref):
    _, bn, p = x_ref.shape
    b = bn * p
    x3 = x_ref[...]
    z2 = jnp.tanh(x3[0:2]).astype(jnp.bfloat16).reshape(2, b)
    t_bf = t_ref[...].astype(jnp.bfloat16)
    tT_bf = jnp.concatenate([t_bf[g] for g in range(bn)], axis=1)
    ones = jnp.ones((1, b), jnp.bfloat16)
    cat = jnp.concatenate([tT_bf, z2, ones], axis=0)
    W1a_bf = W1a_ref[...].astype(jnp.bfloat16)
    hT = jax.lax.dot_general(W1a_bf, cat, (((1,), (0,)), ((), ())),
                             preferred_element_type=jnp.float32)
    hT = jnp.maximum(hT, 0.0)
    pT = (W2T_ref[...] @ hT).reshape(10, bn, p) + b2_ref[...].reshape(10, 1, 1)
    dT = jax.nn.softplus(pT) + 1e-4
    xL1 = -dT[1:2]
    xL2 = xL1 - dT[0:1]
    xL3 = xL2 - 10000.0
    xR1 = dT[2:3]
    xR2 = xR1 + dT[3:4]
    xR3 = xR2 + 10000.0
    yL1 = -dT[5:6]
    yL2 = yL1 - dT[4:5]
    yL3 = yL2 - 20000.0 * dT[8:9]
    yR1 = dT[6:7]
    yR2 = yR1 + dT[7:8]
    yR3 = yR2 + 20000.0 * dT[9:10]
    ax = jnp.concatenate([xL3, xL2, xL1, xR1, xR2, xR3], axis=0)
    ay = jnp.concatenate([yL3, yL2, yL1, yR1, yR2, yR3], axis=0)
    qx = jnp.clip(x3[2:3], ax[0:1] * 0.99, ax[5:6] * 0.99)
    xl = ax[0:5]
    xr = ax[1:6]
    yl = ay[0:5]
    yr = ay[1:6]
    sel = ((qx >= xl) & (qx < xr)).astype(jnp.float32)
    xl_s = jnp.sum(xl * sel, axis=0, keepdims=True)
    xr_s = jnp.sum(xr * sel, axis=0, keepdims=True)
    yl_s = jnp.sum(yl * sel, axis=0, keepdims=True)
    yr_s = jnp.sum(yr * sel, axis=0, keepdims=True)
    gy = (yr_s - yl_s) / (xr_s - xl_s) * (qx - xl_s) + yl_s
    out_ref[...] = jnp.concatenate(
        [x3[0:2] * mask_ref[0:2].reshape(2, 1, 1), gy], axis=0)


def kernel(x, t_feat, mask, W1, b1, W2, b2):
    n, p, _ = x.shape
    rows = n * p
    blk = 32768
    bn = blk // p
    grid = rows // blk
    W1a = jnp.concatenate([W1[2:, :].T, W1[0:2, :].T, b1.reshape(128, 1)],
                          axis=1)
    x3 = jnp.transpose(x, (2, 0, 1))
    tt = jnp.transpose(t_feat, (0, 2, 1))
    const = lambda i: (0, 0)
    out = pl.pallas_call(
        _body,
        grid=(grid,),
        in_specs=[
            pl.BlockSpec((3, bn, p), lambda i: (0, i, 0)),
            pl.BlockSpec((bn, 64, p), lambda i: (i, 0, 0)),
            pl.BlockSpec((3, 1), const),
            pl.BlockSpec((128, 67), const),
            pl.BlockSpec((10, 128), const),
            pl.BlockSpec((10, 1), const),
        ],
        out_specs=pl.BlockSpec((3, bn, p), lambda i: (0, i, 0)),
        out_shape=jax.ShapeDtypeStruct((3, n, p), jnp.float32),
    )(x3, tt, mask.reshape(3, 1), W1a,
      W2.T, b2.reshape(10, 1))
    return jnp.transpose(out, (1, 2, 0))
